# Initial kernel scaffold; baseline (speedup 1.0000x reference)
import functools

import jax
import jax.numpy as jnp
from jax import lax
from jax.experimental import pallas as pl
from jax.experimental.pallas import tpu as pltpu
from jax.experimental.pallas import tpu_sc as plsc

NTOK = 4 * 8192
D = 1024
NC, NS = 2, 16
NW = NC * NS
PER_W = NTOK // NW
ROWS = 2048
NB = NTOK // ROWS


def _make_flag_gather():
    mesh = plsc.VectorSubcoreMesh(core_axis_name="c", subcore_axis_name="s")

    @functools.partial(
        pl.kernel,
        mesh=mesh,
        out_type=jax.ShapeDtypeStruct((NTOK,), jnp.int32),
        scratch_types=[
            pltpu.VMEM((PER_W,), jnp.int32),
            pltpu.VMEM((PER_W,), jnp.int32),
            pltpu.SemaphoreType.DMA,
        ],
    )
    def gather_flags(ids_hbm, table_hbm, out_hbm, idx_v, flags_v, sem):
        wid = lax.axis_index("s") * NC + lax.axis_index("c")
        base = wid * PER_W
        pltpu.sync_copy(ids_hbm.at[pl.ds(base, PER_W)], idx_v)
        pltpu.async_copy(table_hbm.at[idx_v], flags_v, sem).wait()
        pltpu.sync_copy(flags_v, out_hbm.at[pl.ds(base, PER_W)])

    return gather_flags


_flag_gather = _make_flag_gather()


def _tc_body(f_ref, w_ref, e_ref, o_ref):
    f = f_ref[...].astype(jnp.float32)
    w0 = w_ref[0:1, :]
    w1 = w_ref[1:2, :]
    o_ref[...] = e_ref[...] + (w0 + f * (w1 - w0))


def kernel(subword_embeds, token_ids, is_continuation, cont_emb_weight):
    vocab = is_continuation.shape[0] - 1
    ids = jnp.minimum(token_ids, vocab).astype(jnp.int32).reshape(NTOK)
    table = is_continuation.astype(jnp.int32)

    flags = _flag_gather(ids, table)

    e2d = subword_embeds.reshape(NTOK, D)
    out = pl.pallas_call(
        _tc_body,
        grid=(NB,),
        in_specs=[
            pl.BlockSpec((ROWS, 1), lambda i: (i, 0)),
            pl.BlockSpec((2, D), lambda i: (0, 0)),
            pl.BlockSpec((ROWS, D), lambda i: (i, 0)),
        ],
        out_specs=pl.BlockSpec((ROWS, D), lambda i: (i, 0)),
        out_shape=jax.ShapeDtypeStruct((NTOK, D), jnp.float32),
    )(flags.reshape(NTOK, 1), cont_emb_weight.astype(jnp.float32), e2d)
    return out.reshape(subword_embeds.shape)


# speedup vs baseline: 2.3519x; 2.3519x over previous
import functools

import jax
import jax.numpy as jnp
from jax import lax
from jax.experimental import pallas as pl
from jax.experimental.pallas import tpu as pltpu
from jax.experimental.pallas import tpu_sc as plsc

NTOK = 4 * 8192
D = 1024
NC, NS = 2, 16
NW = NC * NS
PER_W = NTOK // NW
ROWS = 2048
NB = NTOK // ROWS


@functools.lru_cache(maxsize=1)
def _make_flag_gather():
    mesh = plsc.VectorSubcoreMesh(core_axis_name="c", subcore_axis_name="s")

    @functools.partial(
        pl.kernel,
        mesh=mesh,
        out_type=jax.ShapeDtypeStruct((NTOK,), jnp.int32),
        scratch_types=[
            pltpu.VMEM((PER_W,), jnp.int32),
            pltpu.VMEM((PER_W,), jnp.int32),
            pltpu.SemaphoreType.DMA,
        ],
    )
    def gather_flags(ids_hbm, table_hbm, out_hbm, idx_v, flags_v, sem):
        wid = lax.axis_index("s") * NC + lax.axis_index("c")
        base = wid * PER_W
        pltpu.sync_copy(ids_hbm.at[pl.ds(base, PER_W)], idx_v)
        pltpu.async_copy(table_hbm.at[idx_v], flags_v, sem).wait()
        pltpu.sync_copy(flags_v, out_hbm.at[pl.ds(base, PER_W)])

    return gather_flags


def _tc_body(f_ref, w_ref, e_ref, o_ref):
    f = f_ref[...].astype(jnp.float32)
    w0 = w_ref[0:1, :]
    w1 = w_ref[1:2, :]
    o_ref[...] = e_ref[...] + (w0 + f * (w1 - w0))


def kernel(subword_embeds, token_ids, is_continuation, cont_emb_weight):
    vocab = is_continuation.shape[0] - 1
    ids = jnp.minimum(token_ids, vocab).astype(jnp.int32).reshape(NTOK)
    table = is_continuation.astype(jnp.int32)

    flags = _make_flag_gather()(ids, table)

    e2d = subword_embeds.reshape(NTOK, D)
    out = pl.pallas_call(
        _tc_body,
        grid=(NB,),
        in_specs=[
            pl.BlockSpec((ROWS, 1), lambda i: (i, 0)),
            pl.BlockSpec((2, D), lambda i: (0, 0)),
            pl.BlockSpec((ROWS, D), lambda i: (i, 0)),
        ],
        out_specs=pl.BlockSpec((ROWS, D), lambda i: (i, 0)),
        out_shape=jax.ShapeDtypeStruct((NTOK, D), jnp.float32),
    )(flags.reshape(NTOK, 1), cont_emb_weight.astype(jnp.float32), e2d)
    return out.reshape(subword_embeds.shape)
